# baseline (speedup 1.0000x reference)
import functools

import jax
import jax.numpy as jnp
from jax import lax
from jax.experimental import pallas as pl
from jax.experimental.pallas import tpu as pltpu
from jax.experimental.pallas import tpu_sc as plsc

_DIM = 32
_NW = 32
_CHUNK = 128
_NBUF = 4
_DETILE_BLK = 256
_ROWS_W = 512
_PIECE = 128


def _detile(idxs):
    b, s, two = idxs.shape
    w = s * two

    def body(x_ref, o_ref):
        x = x_ref[...].astype(jnp.float32)
        srow = lax.broadcasted_iota(jnp.int32, (s, 128), 0)
        ccol = lax.broadcasted_iota(jnp.int32, (s, 128), 1)
        acc = jnp.zeros((_DETILE_BLK, 128), jnp.float32)
        for t in range(two):
            xt = jnp.sum(
                x * (lax.broadcasted_iota(jnp.int32, (1, 1, two), 2) == t),
                axis=2)
            st = (ccol == 2 * srow + t).astype(jnp.float32)
            acc = acc + jax.lax.dot_general(
                xt, st, (((1,), (0,)), ((), ())),
                preferred_element_type=jnp.float32,
                precision=lax.Precision.HIGHEST)
        o_ref[...] = acc.astype(jnp.int32)

    return pl.pallas_call(
        body,
        grid=(b // _DETILE_BLK,),
        in_specs=[pl.BlockSpec((_DETILE_BLK, s, two), lambda i: (i, 0, 0))],
        out_specs=pl.BlockSpec((_DETILE_BLK, 128), lambda i: (i, 0)),
        out_shape=jax.ShapeDtypeStruct((b, 128), jnp.int32),
    )(idxs)


def _poe_pallas(idx128, emb):
    n = idx128.shape[0] * 50
    per_w = n // _NW
    n_chunks = per_w // _CHUNK
    groups = _CHUNK // 16

    mesh = plsc.VectorSubcoreMesh(
        core_axis_name="c", subcore_axis_name="s", num_cores=2, num_subcores=16
    )

    @functools.partial(
        pl.kernel,
        out_type=jax.ShapeDtypeStruct((n,), jnp.float32),
        mesh=mesh,
        compiler_params=pltpu.CompilerParams(
            needs_layout_passes=False, use_tc_tiling_on_sc=False),
        scratch_types=[
            pltpu.VMEM((2 * per_w + 16,), jnp.int32),
            pltpu.VMEM((_PIECE, 128), jnp.int32),
        ] + [pltpu.VMEM((2 * _CHUNK, _DIM), jnp.float32)] * _NBUF
          + [pltpu.VMEM((_CHUNK,), jnp.float32)] * _NBUF
          + [pltpu.SemaphoreType.DMA] * (2 * _NBUF),
    )
    def run(idx_hbm, emb_hbm, out_hbm, idx_v, raw_v, *rest):
        bufs = rest[:_NBUF]
        obufs = rest[_NBUF:2 * _NBUF]
        sems = rest[2 * _NBUF:3 * _NBUF]
        osems = rest[3 * _NBUF:]
        wid = lax.axis_index("s") * 2 + lax.axis_index("c")
        base = wid * per_w
        lanes = lax.iota(jnp.int32, 16)

        gdn = lax.GatherDimensionNumbers(
            offset_dims=(), collapsed_slice_dims=(0,), start_index_map=(0,))

        def permute(v, idx):
            return lax.gather(
                v, idx[:, None], gdn, (1,),
                mode=lax.GatherScatterMode.PROMISE_IN_BOUNDS)

        def rot(v, k):
            return permute(v, (lanes + k) & 15)

        bitrev = (((lanes & 1) << 3) | ((lanes & 2) << 1)
                  | ((lanes & 4) >> 1) | ((lanes & 8) >> 3))
        m8 = lanes < 8
        m4 = (lanes & 7) < 4
        m2 = (lanes & 3) < 2
        m1 = (lanes & 1) < 1

        for piece in range(_ROWS_W // _PIECE):
            pltpu.sync_copy(
                idx_hbm.at[pl.ds(wid * _ROWS_W + piece * _PIECE, _PIECE)],
                raw_v)

            def compact_body(r, c2, piece=piece):
                out0 = (piece * _PIECE + r) * 100
                for o in range(0, 112, 16):
                    idx_v[pl.ds(out0 + o, 16)] = raw_v[r, pl.ds(o, 16)]
                return c2
            lax.fori_loop(0, _PIECE, compact_body, 0)

        def fire(g, r, sem):
            src = pl.ds(g * (2 * _CHUNK), 2 * _CHUNK)
            pltpu.async_copy(emb_hbm.at[idx_v.at[src]], r, sem)

        def drain(r, sem):
            pltpu.make_async_copy(
                emb_hbm.at[idx_v.at[pl.ds(0, 2 * _CHUNK)]], r, sem).wait()

        def compute(g, r, ob):
            def group_body(gi, c2):
                row0 = gi * 32
                us = []
                for j in range(16):
                    e1 = row0 + 2 * j
                    a = r[e1, pl.ds(0, 16)]
                    b = r[e1, pl.ds(16, 16)]
                    c = r[e1 + 1, pl.ds(0, 16)]
                    d = r[e1 + 1, pl.ds(16, 16)]
                    us.append(jnp.maximum(a - c, 0.0) + jnp.maximum(b - d, 0.0))
                xs = [u + rot(u, 8) for u in us]
                ys = [jnp.where(m8, xs[2 * k], xs[2 * k + 1]) for k in range(8)]
                zs = [y + rot(y, 4) for y in ys]
                ws = [jnp.where(m4, zs[2 * k], rot(zs[2 * k + 1], -4))
                      for k in range(4)]
                ts = [w + rot(w, 2) for w in ws]
                vs = [jnp.where(m2, ts[2 * k], rot(ts[2 * k + 1], -2))
                      for k in range(2)]
                ss = [v + rot(v, 1) for v in vs]
                s = jnp.where(m1, ss[0], rot(ss[1], -1))
                ob[pl.ds(gi * 16, 16)] = -permute(s, bitrev)
                return c2
            lax.fori_loop(0, groups, group_body, 0)

        for b in range(_NBUF - 1):
            fire(b, bufs[b], sems[b])

        def ring_body(i, carry):
            g0 = i * _NBUF
            for b in range(_NBUF):
                g = g0 + b
                ahead = g + _NBUF - 1
                ba = (b + _NBUF - 1) % _NBUF

                @pl.when(ahead < n_chunks)
                def _(ahead=ahead, ba=ba):
                    fire(ahead, bufs[ba], sems[ba])

                drain(bufs[b], sems[b])

                @pl.when(g >= _NBUF)
                def _(b=b):
                    pltpu.make_async_copy(
                        obufs[b], out_hbm.at[pl.ds(base, _CHUNK)],
                        osems[b]).wait()

                compute(g, bufs[b], obufs[b])
                pltpu.async_copy(
                    obufs[b], out_hbm.at[pl.ds(base + g * _CHUNK, _CHUNK)],
                    osems[b])
            return carry

        lax.fori_loop(0, n_chunks // _NBUF, ring_body, 0)
        for b in range(_NBUF):
            pltpu.make_async_copy(
                obufs[b], out_hbm.at[pl.ds(base, _CHUNK)], osems[b]).wait()

    return run(idx128, emb)


def kernel(idxs, emb):
    b, s, _ = idxs.shape
    out = _poe_pallas(_detile(idxs), emb)
    return out.reshape(b, s)


# speedup vs baseline: 1.6174x; 1.6174x over previous
import functools

import jax
import jax.numpy as jnp
from jax import lax
from jax.experimental import pallas as pl
from jax.experimental.pallas import tpu as pltpu
from jax.experimental.pallas import tpu_sc as plsc

_DIM = 32
_NW = 32
_CHUNK = 256
_NBUF = 4


def _poe_pallas(idx1, idx2, emb):
    n = idx1.shape[0]
    per_w = n // _NW
    n_chunks = per_w // _CHUNK
    groups = _CHUNK // 16

    mesh = plsc.VectorSubcoreMesh(
        core_axis_name="c", subcore_axis_name="s", num_cores=2, num_subcores=16
    )

    @functools.partial(
        pl.kernel,
        out_type=jax.ShapeDtypeStruct((n,), jnp.float32),
        mesh=mesh,
        compiler_params=pltpu.CompilerParams(
            needs_layout_passes=False, use_tc_tiling_on_sc=False),
        scratch_types=[
            pltpu.VMEM((2 * per_w,), jnp.int32),
        ] + [pltpu.VMEM((2 * _CHUNK, _DIM), jnp.float32)] * _NBUF
          + [pltpu.VMEM((_CHUNK,), jnp.float32)] * _NBUF
          + [pltpu.SemaphoreType.DMA] * (2 * _NBUF),
    )
    def run(idx1_hbm, idx2_hbm, emb_hbm, out_hbm, idx_v, *rest):
        bufs = rest[:_NBUF]
        obufs = rest[_NBUF:2 * _NBUF]
        sems = rest[2 * _NBUF:3 * _NBUF]
        osems = rest[3 * _NBUF:]
        wid = lax.axis_index("s") * 2 + lax.axis_index("c")
        base = wid * per_w
        lanes = lax.iota(jnp.int32, 16)

        gdn = lax.GatherDimensionNumbers(
            offset_dims=(), collapsed_slice_dims=(0,), start_index_map=(0,))

        def permute(v, idx):
            return lax.gather(
                v, idx[:, None], gdn, (1,),
                mode=lax.GatherScatterMode.PROMISE_IN_BOUNDS)

        def rot(v, k):
            return permute(v, (lanes + k) & 15)

        bitrev = (((lanes & 1) << 3) | ((lanes & 2) << 1)
                  | ((lanes & 4) >> 1) | ((lanes & 8) >> 3))
        m8 = lanes < 8
        m4 = (lanes & 7) < 4
        m2 = (lanes & 3) < 2
        m1 = (lanes & 1) < 1

        pltpu.sync_copy(idx1_hbm.at[pl.ds(base, per_w)],
                        idx_v.at[pl.ds(0, per_w)])
        pltpu.sync_copy(idx2_hbm.at[pl.ds(base, per_w)],
                        idx_v.at[pl.ds(per_w, per_w)])

        def fire(g, r, sem):
            src1 = pl.ds(g * _CHUNK, _CHUNK)
            src2 = pl.ds(per_w + g * _CHUNK, _CHUNK)
            pltpu.async_copy(emb_hbm.at[idx_v.at[src1]],
                             r.at[pl.ds(0, _CHUNK)], sem)
            pltpu.async_copy(emb_hbm.at[idx_v.at[src2]],
                             r.at[pl.ds(_CHUNK, _CHUNK)], sem)

        def drain(r, sem):
            for j in range(2):
                dst = pl.ds(j * _CHUNK, _CHUNK)
                pltpu.make_async_copy(
                    emb_hbm.at[idx_v.at[pl.ds(0, _CHUNK)]], r.at[dst], sem).wait()

        def compute(g, r, ob):
            def group_body(gi, c2):
                row0 = gi * 16
                us = []
                for j in range(16):
                    e1 = row0 + j
                    e2 = _CHUNK + e1
                    a = r[e1, pl.ds(0, 16)]
                    b = r[e1, pl.ds(16, 16)]
                    c = r[e2, pl.ds(0, 16)]
                    d = r[e2, pl.ds(16, 16)]
                    us.append(jnp.maximum(a - c, 0.0) + jnp.maximum(b - d, 0.0))
                xs = [u + rot(u, 8) for u in us]
                ys = [jnp.where(m8, xs[2 * k], xs[2 * k + 1]) for k in range(8)]
                zs = [y + rot(y, 4) for y in ys]
                ws = [jnp.where(m4, zs[2 * k], rot(zs[2 * k + 1], -4))
                      for k in range(4)]
                ts = [w + rot(w, 2) for w in ws]
                vs = [jnp.where(m2, ts[2 * k], rot(ts[2 * k + 1], -2))
                      for k in range(2)]
                ss = [v + rot(v, 1) for v in vs]
                s = jnp.where(m1, ss[0], rot(ss[1], -1))
                ob[pl.ds(gi * 16, 16)] = -permute(s, bitrev)
                return c2
            lax.fori_loop(0, groups, group_body, 0)

        for b in range(_NBUF - 1):
            fire(b, bufs[b], sems[b])

        def ring_body(i, carry):
            g0 = i * _NBUF
            for b in range(_NBUF):
                g = g0 + b
                ahead = g + _NBUF - 1
                ba = (b + _NBUF - 1) % _NBUF

                @pl.when(ahead < n_chunks)
                def _(ahead=ahead, ba=ba):
                    fire(ahead, bufs[ba], sems[ba])

                drain(bufs[b], sems[b])

                @pl.when(g >= _NBUF)
                def _(b=b):
                    pltpu.make_async_copy(
                        obufs[b], out_hbm.at[pl.ds(base, _CHUNK)],
                        osems[b]).wait()

                compute(g, bufs[b], obufs[b])
                pltpu.async_copy(
                    obufs[b], out_hbm.at[pl.ds(base + g * _CHUNK, _CHUNK)],
                    osems[b])
            return carry

        lax.fori_loop(0, n_chunks // _NBUF, ring_body, 0)
        for b in range(_NBUF):
            pltpu.make_async_copy(
                obufs[b], out_hbm.at[pl.ds(base, _CHUNK)], osems[b]).wait()

    return run(idx1, idx2, emb)


def kernel(idxs, emb):
    b, s, _ = idxs.shape
    flat = idxs.reshape(-1, 2)
    out = _poe_pallas(flat[:, 0], flat[:, 1], emb)
    return out.reshape(b, s)


# speedup vs baseline: 1.6400x; 1.0140x over previous
import functools

import jax
import jax.numpy as jnp
from jax import lax
from jax.experimental import pallas as pl
from jax.experimental.pallas import tpu as pltpu
from jax.experimental.pallas import tpu_sc as plsc

_DIM = 32
_NW = 32
_CHUNK = 256
_NBUF = 4


def _poe_pallas(idx1, idx2, emb):
    n = idx1.shape[0]
    per_w = n // _NW
    n_chunks = per_w // _CHUNK
    groups = _CHUNK // 16

    mesh = plsc.VectorSubcoreMesh(
        core_axis_name="c", subcore_axis_name="s", num_cores=2, num_subcores=16
    )

    @functools.partial(
        pl.kernel,
        out_type=jax.ShapeDtypeStruct((n,), jnp.float32),
        mesh=mesh,
        compiler_params=pltpu.CompilerParams(
            needs_layout_passes=False, use_tc_tiling_on_sc=False),
        scratch_types=[
            pltpu.VMEM((2 * per_w,), jnp.int32),
        ] + [pltpu.VMEM((2 * _CHUNK, _DIM), jnp.float32)] * _NBUF
          + [pltpu.VMEM((_CHUNK,), jnp.float32)] * _NBUF
          + [pltpu.SemaphoreType.DMA] * (2 * _NBUF),
    )
    def run(idx1_hbm, idx2_hbm, emb_hbm, out_hbm, idx_v, *rest):
        bufs = rest[:_NBUF]
        obufs = rest[_NBUF:2 * _NBUF]
        sems = rest[2 * _NBUF:3 * _NBUF]
        osems = rest[3 * _NBUF:]
        wid = lax.axis_index("s") * 2 + lax.axis_index("c")
        base = wid * per_w
        lanes = lax.iota(jnp.int32, 16)

        gdn = lax.GatherDimensionNumbers(
            offset_dims=(), collapsed_slice_dims=(0,), start_index_map=(0,))

        def permute(v, idx):
            return lax.gather(
                v, idx[:, None], gdn, (1,),
                mode=lax.GatherScatterMode.PROMISE_IN_BOUNDS)

        def rot(v, k):
            return permute(v, (lanes + k) & 15)

        bitrev = (((lanes & 1) << 3) | ((lanes & 2) << 1)
                  | ((lanes & 4) >> 1) | ((lanes & 8) >> 3))
        m8 = lanes < 8
        m4 = (lanes & 7) < 4
        m2 = (lanes & 3) < 2
        m1 = (lanes & 1) < 1

        pltpu.sync_copy(idx1_hbm.at[pl.ds(base, per_w)],
                        idx_v.at[pl.ds(0, per_w)])
        pltpu.sync_copy(idx2_hbm.at[pl.ds(base, per_w)],
                        idx_v.at[pl.ds(per_w, per_w)])

        def fire(g, r, sem):
            src1 = pl.ds(g * _CHUNK, _CHUNK)
            src2 = pl.ds(per_w + g * _CHUNK, _CHUNK)
            pltpu.async_copy(emb_hbm.at[idx_v.at[src1]],
                             r.at[pl.ds(0, _CHUNK)], sem)
            pltpu.async_copy(emb_hbm.at[idx_v.at[src2]],
                             r.at[pl.ds(_CHUNK, _CHUNK)], sem)

        def drain(r, sem):
            for j in range(2):
                dst = pl.ds(j * _CHUNK, _CHUNK)
                pltpu.make_async_copy(
                    emb_hbm.at[idx_v.at[pl.ds(0, _CHUNK)]], r.at[dst], sem).wait()

        def compute(g, r, ob):
            def group_body(gi, c2):
                row0 = gi * 16
                us = []
                for j in range(16):
                    e1 = row0 + j
                    e2 = _CHUNK + e1
                    a = r[e1, pl.ds(0, 16)]
                    b = r[e1, pl.ds(16, 16)]
                    c = r[e2, pl.ds(0, 16)]
                    d = r[e2, pl.ds(16, 16)]
                    us.append(jnp.maximum(a - c, 0.0) + jnp.maximum(b - d, 0.0))
                xs = [u + rot(u, 8) for u in us]
                ys = [jnp.where(m8, xs[2 * k], xs[2 * k + 1]) for k in range(8)]
                zs = [y + rot(y, 4) for y in ys]
                ws = [jnp.where(m4, zs[2 * k], rot(zs[2 * k + 1], -4))
                      for k in range(4)]
                ts = [w + rot(w, 2) for w in ws]
                vs = [jnp.where(m2, ts[2 * k], rot(ts[2 * k + 1], -2))
                      for k in range(2)]
                ss = [v + rot(v, 1) for v in vs]
                s = jnp.where(m1, ss[0], rot(ss[1], -1))
                ob[pl.ds(gi * 16, 16)] = -permute(s, bitrev)
                return c2
            lax.fori_loop(0, groups, group_body, 0)

        for b in range(_NBUF - 1):
            fire(b, bufs[b], sems[b])

        def ring_body(i, carry):
            g0 = i * _NBUF
            for b in range(_NBUF):
                g = g0 + b
                ahead = g + _NBUF - 1
                ba = (b + _NBUF - 1) % _NBUF

                @pl.when(ahead < n_chunks)
                def _(ahead=ahead, ba=ba):
                    fire(ahead, bufs[ba], sems[ba])

                drain(bufs[b], sems[b])

                @pl.when(g >= _NBUF)
                def _(b=b):
                    pltpu.make_async_copy(
                        obufs[b], out_hbm.at[pl.ds(base, _CHUNK)],
                        osems[b]).wait()

                compute(g, bufs[b], obufs[b])
                pltpu.async_copy(
                    obufs[b], out_hbm.at[pl.ds(base + g * _CHUNK, _CHUNK)],
                    osems[b])
            return carry

        lax.fori_loop(0, n_chunks // _NBUF, ring_body, 0)
        for b in range(_NBUF):
            pltpu.make_async_copy(
                obufs[b], out_hbm.at[pl.ds(base, _CHUNK)], osems[b]).wait()

    return run(idx1, idx2, emb)


def kernel(idxs, emb):
    b, s, _ = idxs.shape
    out = _poe_pallas(idxs[:, :, 0].reshape(-1), idxs[:, :, 1].reshape(-1), emb)
    return out.reshape(b, s)
